# Initial kernel scaffold; baseline (speedup 1.0000x reference)
import jax
import jax.numpy as jnp
from jax.experimental import pallas as pl


def kernel(camera_ids, weight):
    raise NotImplementedError("write your pallas kernel here")


# speedup vs baseline: 1.0319x; 1.0319x over previous
import functools

import jax
import jax.numpy as jnp
from jax import lax
from jax.experimental import pallas as pl
from jax.experimental.pallas import tpu as pltpu
from jax.experimental.pallas import tpu_sc as plsc

EMBED_DIM = 32
NUM_CORES = 2
NUM_SUBCORES = 16
NUM_WORKERS = NUM_CORES * NUM_SUBCORES
CHUNK = 512


def _make_gather(batch: int, vocab: int, dim: int):
  assert batch % (NUM_WORKERS * CHUNK) == 0
  b_per_w = batch // NUM_WORKERS
  n_chunks = b_per_w // CHUNK
  mesh = plsc.VectorSubcoreMesh(
      core_axis_name="c",
      subcore_axis_name="s",
      num_cores=NUM_CORES,
      num_subcores=NUM_SUBCORES,
  )

  @functools.partial(
      pl.kernel,
      out_type=jax.ShapeDtypeStruct((batch, dim), jnp.float32),
      mesh=mesh,
      scratch_types=[
          pltpu.VMEM((CHUNK,), jnp.int32),
          pltpu.VMEM((CHUNK, dim), jnp.float32),
          pltpu.SemaphoreType.DMA,
      ],
      compiler_params=pltpu.CompilerParams(use_tc_tiling_on_sc=False),
  )
  def gather_kernel(ids_hbm, w_hbm, out_hbm, idx_v, rows_v, sem):
    wid = lax.axis_index("s") * NUM_CORES + lax.axis_index("c")
    base = wid * b_per_w

    def body(i, carry):
      off = base + i * CHUNK
      pltpu.sync_copy(ids_hbm.at[pl.ds(off, CHUNK)], idx_v)
      pltpu.async_copy(w_hbm.at[idx_v], rows_v, sem).wait()
      pltpu.sync_copy(rows_v, out_hbm.at[pl.ds(off, CHUNK)])
      return carry

    lax.fori_loop(0, n_chunks, body, 0)

  return gather_kernel


def kernel(camera_ids, weight):
  ids = camera_ids.reshape(-1).astype(jnp.int32)
  batch = ids.shape[0]
  vocab, dim = weight.shape
  return _make_gather(batch, vocab, dim)(ids, weight)


# speedup vs baseline: 1.0918x; 1.0581x over previous
import functools

import jax
import jax.numpy as jnp
from jax import lax
from jax.experimental import pallas as pl
from jax.experimental.pallas import tpu as pltpu
from jax.experimental.pallas import tpu_sc as plsc

NUM_CORES = 2
NUM_SUBCORES = 16
NUM_WORKERS = NUM_CORES * NUM_SUBCORES
CHUNK = 1280


def _make_gather(batch: int, dim: int):
  assert batch % (NUM_WORKERS * CHUNK) == 0
  b_per_w = batch // NUM_WORKERS
  n_chunks = b_per_w // CHUNK
  assert n_chunks % 2 == 0
  n_pairs = n_chunks // 2
  mesh = plsc.VectorSubcoreMesh(
      core_axis_name="c",
      subcore_axis_name="s",
      num_cores=NUM_CORES,
      num_subcores=NUM_SUBCORES,
  )

  @functools.partial(
      pl.kernel,
      out_type=jax.ShapeDtypeStruct((batch, dim), jnp.float32),
      mesh=mesh,
      scratch_types=[
          pltpu.VMEM((b_per_w,), jnp.int32),
          pltpu.VMEM((CHUNK, dim), jnp.float32),
          pltpu.VMEM((CHUNK, dim), jnp.float32),
          pltpu.SemaphoreType.DMA,
          pltpu.SemaphoreType.DMA,
      ],
      compiler_params=pltpu.CompilerParams(use_tc_tiling_on_sc=False),
  )
  def gather_kernel(ids_hbm, w_hbm, out_hbm, idx_v, rows0, rows1, g_sem,
                    o_sem):
    wid = lax.axis_index("s") * NUM_CORES + lax.axis_index("c")
    base = wid * b_per_w

    pltpu.sync_copy(ids_hbm.at[pl.ds(base, b_per_w)], idx_v)

    def g_start(j, rows):
      pltpu.async_copy(w_hbm.at[idx_v.at[pl.ds(j * CHUNK, CHUNK)]], rows,
                       g_sem)

    def g_wait(rows):
      pltpu.make_async_copy(w_hbm.at[idx_v.at[pl.ds(0, CHUNK)]], rows,
                            g_sem).wait()

    def o_start(j, rows):
      pltpu.async_copy(rows, out_hbm.at[pl.ds(base + j * CHUNK, CHUNK)], o_sem)

    def o_wait(rows):
      pltpu.make_async_copy(rows, out_hbm.at[pl.ds(base, CHUNK)], o_sem).wait()

    g_start(0, rows0)

    def body(k, carry):
      i = 2 * k
      g_wait(rows0)

      @pl.when(k > 0)
      def _():
        o_wait(rows1)

      g_start(i + 1, rows1)
      o_start(i, rows0)
      g_wait(rows1)
      o_wait(rows0)

      @pl.when(k + 1 < n_pairs)
      def _():
        g_start(i + 2, rows0)

      o_start(i + 1, rows1)
      return carry

    lax.fori_loop(0, n_pairs, body, 0)
    o_wait(rows1)

  return gather_kernel


def kernel(camera_ids, weight):
  ids = camera_ids.reshape(-1).astype(jnp.int32)
  batch = ids.shape[0]
  dim = weight.shape[1]
  return _make_gather(batch, dim)(ids, weight)
